# Initial kernel scaffold; baseline (speedup 1.0000x reference)
import jax
import jax.numpy as jnp
from jax.experimental import pallas as pl


def kernel(patches, token_table, pos_table):
    raise NotImplementedError("write your pallas kernel here")


# speedup vs baseline: 5.1479x; 5.1479x over previous
import functools

import jax
import jax.numpy as jnp
from jax import lax
from jax.experimental import pallas as pl
from jax.experimental.pallas import tpu as pltpu
from jax.experimental.pallas import tpu_sc as plsc

EMBED = 32
TOK_V = 1024
POS_V = 64
BATCH = 4096
SEQ = 128

NC, NS = 2, 16
NW = NC * NS
SEQ_PER_W = BATCH // NW
CHUNK = 16
NCHUNK = SEQ_PER_W // CHUNK


def _prep_body(tok_ref, pos_ref, patch_ref, fused_ref, fidx_ref):
    tok = tok_ref[...]
    pos = pos_ref[...]
    fused_ref[...] = tok[:, None, :] + pos[None, :, :]
    s = lax.broadcasted_iota(jnp.int32, patch_ref.shape, 1)
    fidx_ref[...] = patch_ref[...] * POS_V + jnp.minimum(s, POS_V - 1)


def _tc_prep(token_table, pos_table, patches):
    grid = 8
    return pl.pallas_call(
        _prep_body,
        grid=(grid,),
        in_specs=[
            pl.BlockSpec((TOK_V // grid, EMBED), lambda i: (i, 0)),
            pl.BlockSpec((POS_V, EMBED), lambda i: (0, 0)),
            pl.BlockSpec((BATCH // grid, SEQ), lambda i: (i, 0)),
        ],
        out_specs=[
            pl.BlockSpec((TOK_V // grid, POS_V, EMBED), lambda i: (i, 0, 0)),
            pl.BlockSpec((BATCH // grid, SEQ), lambda i: (i, 0)),
        ],
        out_shape=[
            jax.ShapeDtypeStruct((TOK_V, POS_V, EMBED), jnp.float32),
            jax.ShapeDtypeStruct((BATCH, SEQ), jnp.int32),
        ],
    )(token_table, pos_table, patches)


def _sc_body(fused_hbm, fidx_hbm, out_hbm, idx_v, rows_v, sem):
    wid = lax.axis_index("s") * NC + lax.axis_index("c")
    for c in range(NCHUNK):
        base = wid * SEQ_PER_W + c * CHUNK
        pltpu.sync_copy(fidx_hbm.at[pl.ds(base, CHUNK)], idx_v)
        copies = [
            pltpu.async_copy(fused_hbm.at[idx_v.at[j]], rows_v.at[j], sem)
            for j in range(CHUNK)
        ]
        for cp in copies:
            cp.wait()
        pltpu.sync_copy(rows_v, out_hbm.at[pl.ds(base, CHUNK)])


@functools.partial(
    pl.kernel,
    out_type=jax.ShapeDtypeStruct((BATCH, SEQ, EMBED), jnp.float32),
    mesh=plsc.VectorSubcoreMesh(core_axis_name="c", subcore_axis_name="s"),
    scratch_types=[
        pltpu.VMEM((CHUNK, SEQ), jnp.int32),
        pltpu.VMEM((CHUNK, SEQ, EMBED), jnp.float32),
        pltpu.SemaphoreType.DMA,
    ],
    compiler_params=pltpu.CompilerParams(use_tc_tiling_on_sc=False),
)
def _sc_gather(fused_hbm, fidx_hbm, out_hbm, idx_v, rows_v, sem):
    _sc_body(fused_hbm, fidx_hbm, out_hbm, idx_v, rows_v, sem)


def kernel(patches, token_table, pos_table):
    patches = patches.astype(jnp.int32)
    fused, fidx = _tc_prep(token_table, pos_table, patches)
    fused2d = fused.reshape(TOK_V * POS_V, EMBED)
    return _sc_gather(fused2d, fidx)
